# Initial kernel scaffold; baseline (speedup 1.0000x reference)
import jax
import jax.numpy as jnp
from jax.experimental import pallas as pl


def kernel(x, edge_index, W1, b1, W2, b2):
    raise NotImplementedError("write your pallas kernel here")


# speedup vs baseline: 36.2202x; 36.2202x over previous
import functools

import jax
import jax.numpy as jnp
from jax import lax
from jax.experimental import pallas as pl
from jax.experimental.pallas import tpu as pltpu
from jax.experimental.pallas import tpu_sc as plsc

N = 10000
E = 320000
D = 128

NC = 2
NS = 16
NW = NC * NS
NPAD = 10240
STRIPE = NPAD // NS
JROWS = 80
EROWS = NW * JROWS
E_PAD = EROWS * 128
EXTRA = E_PAD - E

BLK = 640


@functools.lru_cache(maxsize=None)
def _build_sc_kernels():
    mesh = plsc.VectorSubcoreMesh(
        core_axis_name="c", subcore_axis_name="s",
        num_cores=NC, num_subcores=NS)

    @functools.partial(
        pl.kernel,
        out_type=jax.ShapeDtypeStruct((NC * NPAD,), jnp.float32),
        mesh=mesh,
        scratch_types=[
            pltpu.VMEM((JROWS, 128), jnp.int32),
            pltpu.VMEM((128,), jnp.float32),
            pltpu.VMEM_SHARED((NPAD,), jnp.float32),
        ],
    )
    def deg_kernel(dst_hbm, ones_hbm, zrow_hbm, out_hbm, idx_v, ones_v, deg_sh):
        c = lax.axis_index("c")
        s = lax.axis_index("s")
        w = c * NS + s
        r0 = s * STRIPE
        pltpu.sync_copy(zrow_hbm.at[pl.ds(r0, STRIPE)],
                        deg_sh.at[pl.ds(r0, STRIPE)])
        pltpu.sync_copy(ones_hbm, ones_v)
        pltpu.sync_copy(dst_hbm.at[pl.ds(w * JROWS, JROWS)], idx_v)
        plsc.subcore_barrier()

        def step(j, carry):
            pltpu.sync_copy(ones_v, deg_sh.at[idx_v.at[j]], add=True)
            return carry

        lax.fori_loop(0, JROWS, step, 0)
        plsc.subcore_barrier()
        pltpu.sync_copy(deg_sh.at[pl.ds(r0, STRIPE)],
                        out_hbm.at[pl.ds(c * NPAD + r0, STRIPE)])

    @functools.partial(
        pl.kernel,
        out_type=(jax.ShapeDtypeStruct((NC * NPAD, D), jnp.float32),
                  jax.ShapeDtypeStruct((NC * NPAD,), jnp.float32)),
        mesh=mesh,
        scratch_types=[
            pltpu.VMEM((JROWS, 128), jnp.int32),
            pltpu.VMEM((JROWS, 128), jnp.int32),
            pltpu.VMEM((128, D), jnp.float32),
            pltpu.VMEM((128,), jnp.float32),
            pltpu.VMEM_SHARED((NPAD, D), jnp.float32),
            pltpu.VMEM_SHARED((NPAD,), jnp.float32),
            pltpu.SemaphoreType.DMA,
            pltpu.SemaphoreType.DMA,
        ],
    )
    def main_kernel(src_hbm, dst_hbm, hs_hbm, dinv_hbm, zmat_hbm, zrow_hbm,
                    acc_out, csum_out,
                    srcv, dstv, updv, dvalv, acc_sh, csum_sh, sem1, sem2):
        c = lax.axis_index("c")
        s = lax.axis_index("s")
        w = c * NS + s
        r0 = s * STRIPE
        pltpu.sync_copy(zmat_hbm.at[pl.ds(r0, STRIPE)],
                        acc_sh.at[pl.ds(r0, STRIPE)])
        pltpu.sync_copy(zrow_hbm.at[pl.ds(r0, STRIPE)],
                        csum_sh.at[pl.ds(r0, STRIPE)])
        pltpu.sync_copy(src_hbm.at[pl.ds(w * JROWS, JROWS)], srcv)
        pltpu.sync_copy(dst_hbm.at[pl.ds(w * JROWS, JROWS)], dstv)
        plsc.subcore_barrier()

        def step(j, carry):
            g1 = pltpu.async_copy(hs_hbm.at[srcv.at[j]], updv, sem1)
            g2 = pltpu.async_copy(dinv_hbm.at[dstv.at[j]], dvalv, sem2)
            g1.wait()
            g2.wait()
            pltpu.sync_copy(updv, acc_sh.at[dstv.at[j]], add=True)
            pltpu.sync_copy(dvalv, csum_sh.at[srcv.at[j]], add=True)
            return carry

        lax.fori_loop(0, JROWS, step, 0)
        plsc.subcore_barrier()
        pltpu.sync_copy(acc_sh.at[pl.ds(r0, STRIPE)],
                        acc_out.at[pl.ds(c * NPAD + r0, STRIPE)])
        pltpu.sync_copy(csum_sh.at[pl.ds(r0, STRIPE)],
                        csum_out.at[pl.ds(c * NPAD + r0, STRIPE)])

    return deg_kernel, main_kernel


def _hs_body(x_ref, w_ref, d_ref, out_ref):
    h = jnp.dot(x_ref[...], w_ref[...], preferred_element_type=jnp.float32)
    out_ref[...] = h * d_ref[...]


def _hs_call(x_pad, W1, dinv_col):
    return pl.pallas_call(
        _hs_body,
        grid=(NPAD // BLK,),
        in_specs=[
            pl.BlockSpec((BLK, D), lambda i: (i, 0)),
            pl.BlockSpec((D, D), lambda i: (0, 0)),
            pl.BlockSpec((BLK, 1), lambda i: (i, 0)),
        ],
        out_specs=pl.BlockSpec((BLK, D), lambda i: (i, 0)),
        out_shape=jax.ShapeDtypeStruct((NPAD, D), jnp.float32),
    )(x_pad, W1, dinv_col)


def _comb_body(a0_ref, a1_ref, hs_ref, d_ref, c_ref, b1_ref, w2_ref, b2_ref,
               out_ref, vacc_ref):
    i = pl.program_id(0)

    @pl.when(i == 0)
    def _():
        vacc_ref[...] = jnp.zeros((1, D), jnp.float32)

    a = a0_ref[...] + a1_ref[...] + hs_ref[...]
    r = jnp.maximum(d_ref[...] * a + b1_ref[...], 0.0)
    vacc_ref[...] += jnp.sum(c_ref[...] * r, axis=0, keepdims=True)

    @pl.when(i == pl.num_programs(0) - 1)
    def _():
        g = jnp.dot(vacc_ref[...], w2_ref[...],
                    preferred_element_type=jnp.float32)
        out_ref[...] = g * (1.0 / N) + b2_ref[...]


def _comb_call(acc2, hs, dinv_col, cvec_col, b1_row, W2, b2_row):
    nblk = NPAD // BLK
    return pl.pallas_call(
        _comb_body,
        grid=(nblk,),
        in_specs=[
            pl.BlockSpec((BLK, D), lambda i: (i, 0)),
            pl.BlockSpec((BLK, D), lambda i: (i + NPAD // BLK, 0)),
            pl.BlockSpec((BLK, D), lambda i: (i, 0)),
            pl.BlockSpec((BLK, 1), lambda i: (i, 0)),
            pl.BlockSpec((BLK, 1), lambda i: (i, 0)),
            pl.BlockSpec((1, D), lambda i: (0, 0)),
            pl.BlockSpec((D, D), lambda i: (0, 0)),
            pl.BlockSpec((1, D), lambda i: (0, 0)),
        ],
        out_specs=pl.BlockSpec((1, D), lambda i: (0, 0)),
        out_shape=jax.ShapeDtypeStruct((1, D), jnp.float32),
        scratch_shapes=[pltpu.VMEM((1, D), jnp.float32)],
    )(acc2, acc2, hs, dinv_col, cvec_col, b1_row, W2, b2_row)


def kernel(x, edge_index, W1, b1, W2, b2):
    deg_kernel, main_kernel = _build_sc_kernels()

    src = edge_index[0]
    dst = edge_index[1]
    pad_idx = (N + (jnp.arange(EXTRA, dtype=jnp.int32) % (NPAD - N))).astype(
        jnp.int32)
    src2d = jnp.concatenate([src, pad_idx]).reshape(EROWS, 128)
    dst2d = jnp.concatenate([dst, pad_idx]).reshape(EROWS, 128)
    x_pad = jnp.pad(x, ((0, NPAD - N), (0, 0)))

    ones_row = jnp.ones((128,), jnp.float32)
    zrow = jnp.zeros((NPAD,), jnp.float32)
    zmat = jnp.zeros((NPAD, D), jnp.float32)

    deg2 = deg_kernel(dst2d, ones_row, zrow)
    deg = deg2[:NPAD] + deg2[NPAD:] + 1.0
    dinv = lax.rsqrt(deg)

    hs = _hs_call(x_pad, W1, dinv.reshape(NPAD, 1))

    acc2, csum2 = main_kernel(src2d, dst2d, hs, dinv, zmat, zrow)

    csum = csum2[:NPAD] + csum2[NPAD:]
    cvec = dinv * (csum + dinv)
    cvec = jnp.where(jnp.arange(NPAD) < N, cvec, 0.0)

    return _comb_call(acc2, hs, dinv.reshape(NPAD, 1), cvec.reshape(NPAD, 1),
                      b1.reshape(1, D), W2, b2.reshape(1, D))


# speedup vs baseline: 48.4100x; 1.3365x over previous
import functools

import jax
import jax.numpy as jnp
from jax import lax
from jax.experimental import pallas as pl
from jax.experimental.pallas import tpu as pltpu
from jax.experimental.pallas import tpu_sc as plsc

N = 10000
E = 320000
D = 128

NC = 2
NS = 16
NW = NC * NS
NPAD = 10240
STRIPE = NPAD // NS
JROWS = 80
EROWS = NW * JROWS
E_PAD = EROWS * 128
EXTRA = E_PAD - E

EPT = JROWS * 128
HEPT = EPT // 2
CL = 128
HCH = HEPT // CL
CLD = 512
NCHD = EPT // CLD

BLK = 640


@functools.lru_cache(maxsize=None)
def _build_sc_kernels():
    mesh = plsc.VectorSubcoreMesh(
        core_axis_name="c", subcore_axis_name="s",
        num_cores=NC, num_subcores=NS)

    @functools.partial(
        pl.kernel,
        out_type=jax.ShapeDtypeStruct((NC * NPAD,), jnp.float32),
        mesh=mesh,
        scratch_types=[
            pltpu.VMEM((EPT,), jnp.int32),
            pltpu.VMEM((CLD,), jnp.float32),
            pltpu.VMEM_SHARED((NPAD,), jnp.float32),
            pltpu.SemaphoreType.DMA,
        ],
    )
    def deg_kernel(dst_hbm, ones_hbm, zrow_hbm, out_hbm,
                   idx_v, ones_v, deg_sh, sem):
        c = lax.axis_index("c")
        s = lax.axis_index("s")
        w = c * NS + s
        r0 = s * STRIPE
        pltpu.sync_copy(ones_hbm, ones_v)
        pltpu.sync_copy(dst_hbm.at[pl.ds(w * EPT, EPT)], idx_v)
        pltpu.sync_copy(zrow_hbm.at[pl.ds(r0, STRIPE)],
                        deg_sh.at[pl.ds(r0, STRIPE)])
        plsc.subcore_barrier()

        def fire(k, carry):
            pltpu.async_copy(
                ones_v, deg_sh.at[idx_v.at[pl.ds(k * CLD, CLD)]], sem,
                add=True)
            return carry

        lax.fori_loop(0, NCHD, fire, 0)

        def drain(k, carry):
            pltpu.make_async_copy(
                ones_v, deg_sh.at[idx_v.at[pl.ds(k * CLD, CLD)]], sem).wait()
            return carry

        lax.fori_loop(0, NCHD, drain, 0)
        plsc.subcore_barrier()
        pltpu.sync_copy(deg_sh.at[pl.ds(r0, STRIPE)],
                        out_hbm.at[pl.ds(c * NPAD + r0, STRIPE)])

    @functools.partial(
        pl.kernel,
        out_type=(jax.ShapeDtypeStruct((NC * NPAD, D), jnp.float32),
                  jax.ShapeDtypeStruct((NC * NPAD,), jnp.float32)),
        mesh=mesh,
        scratch_types=[
            pltpu.VMEM((HEPT,), jnp.int32),
            pltpu.VMEM((HEPT,), jnp.int32),
            pltpu.VMEM((CL, D), jnp.float32),
            pltpu.VMEM((CL, D), jnp.float32),
            pltpu.VMEM((CL,), jnp.float32),
            pltpu.VMEM((CL,), jnp.float32),
            pltpu.VMEM_SHARED((NPAD, D), jnp.float32),
            pltpu.VMEM_SHARED((NPAD,), jnp.float32),
            pltpu.SemaphoreType.DMA,
            pltpu.SemaphoreType.DMA,
            pltpu.SemaphoreType.DMA,
            pltpu.SemaphoreType.DMA,
        ],
    )
    def main_kernel(src_hbm, dst_hbm, hs_hbm, dinv_hbm, zmat_hbm, zrow_hbm,
                    acc_out, csum_out,
                    srcv, dstv, upd0, upd1, dv0, dv1, acc_sh, csum_sh,
                    su0, su1, sd0, sd1):
        c = lax.axis_index("c")
        s = lax.axis_index("s")
        w = c * NS + s
        r0 = s * STRIPE
        ubufs = (upd0, upd1)
        dbufs = (dv0, dv1)
        usems = (su0, su1)
        dsems = (sd0, sd1)

        def pair(i, carry):
            for b in range(2):
                k = i * 2 + b
                off = k * CL
                src_sl = srcv.at[pl.ds(off, CL)]
                dst_sl = dstv.at[pl.ds(off, CL)]
                pltpu.make_async_copy(
                    hs_hbm.at[src_sl], ubufs[b], usems[b]).wait()
                pltpu.make_async_copy(
                    dinv_hbm.at[dst_sl], dbufs[b], dsems[b]).wait()
                pltpu.sync_copy(ubufs[b], acc_sh.at[dst_sl], add=True)
                pltpu.sync_copy(dbufs[b], csum_sh.at[src_sl], add=True)

                @pl.when(k + 2 < HCH)
                def _():
                    off2 = (k + 2) * CL
                    pltpu.async_copy(
                        hs_hbm.at[srcv.at[pl.ds(off2, CL)]],
                        ubufs[b], usems[b])
                    pltpu.async_copy(
                        dinv_hbm.at[dstv.at[pl.ds(off2, CL)]],
                        dbufs[b], dsems[b])
            return carry

        for h in range(2):
            base = w * EPT + h * HEPT
            pltpu.sync_copy(src_hbm.at[pl.ds(base, HEPT)], srcv)
            pltpu.sync_copy(dst_hbm.at[pl.ds(base, HEPT)], dstv)
            for b in range(2):
                pltpu.async_copy(
                    hs_hbm.at[srcv.at[pl.ds(b * CL, CL)]],
                    ubufs[b], usems[b])
                pltpu.async_copy(
                    dinv_hbm.at[dstv.at[pl.ds(b * CL, CL)]],
                    dbufs[b], dsems[b])
            if h == 0:
                pltpu.sync_copy(zmat_hbm.at[pl.ds(r0, STRIPE)],
                                acc_sh.at[pl.ds(r0, STRIPE)])
                pltpu.sync_copy(zrow_hbm.at[pl.ds(r0, STRIPE)],
                                csum_sh.at[pl.ds(r0, STRIPE)])
                plsc.subcore_barrier()
            lax.fori_loop(0, HCH // 2, pair, 0)
        plsc.subcore_barrier()
        pltpu.sync_copy(acc_sh.at[pl.ds(r0, STRIPE)],
                        acc_out.at[pl.ds(c * NPAD + r0, STRIPE)])
        pltpu.sync_copy(csum_sh.at[pl.ds(r0, STRIPE)],
                        csum_out.at[pl.ds(c * NPAD + r0, STRIPE)])

    return deg_kernel, main_kernel


def _mm_body(x_ref, w_ref, out_ref):
    out_ref[...] = jnp.dot(x_ref[...], w_ref[...],
                           preferred_element_type=jnp.float32)


def _mm_call(x_pad, W1):
    return pl.pallas_call(
        _mm_body,
        grid=(NPAD // BLK,),
        in_specs=[
            pl.BlockSpec((BLK, D), lambda i: (i, 0)),
            pl.BlockSpec((D, D), lambda i: (0, 0)),
        ],
        out_specs=pl.BlockSpec((BLK, D), lambda i: (i, 0)),
        out_shape=jax.ShapeDtypeStruct((NPAD, D), jnp.float32),
    )(x_pad, W1)


def _scale_body(h_ref, d_ref, out_ref):
    out_ref[...] = h_ref[...] * d_ref[...]


def _scale_call(h, dinv_col):
    return pl.pallas_call(
        _scale_body,
        grid=(NPAD // BLK,),
        in_specs=[
            pl.BlockSpec((BLK, D), lambda i: (i, 0)),
            pl.BlockSpec((BLK, 1), lambda i: (i, 0)),
        ],
        out_specs=pl.BlockSpec((BLK, D), lambda i: (i, 0)),
        out_shape=jax.ShapeDtypeStruct((NPAD, D), jnp.float32),
    )(h, dinv_col)


def _comb_body(a0_ref, a1_ref, hs_ref, d_ref, c_ref, b1_ref, w2_ref, b2_ref,
               out_ref, vacc_ref):
    i = pl.program_id(0)

    @pl.when(i == 0)
    def _():
        vacc_ref[...] = jnp.zeros((1, D), jnp.float32)

    a = a0_ref[...] + a1_ref[...] + hs_ref[...]
    r = jnp.maximum(d_ref[...] * a + b1_ref[...], 0.0)
    vacc_ref[...] += jnp.sum(c_ref[...] * r, axis=0, keepdims=True)

    @pl.when(i == pl.num_programs(0) - 1)
    def _():
        g = jnp.dot(vacc_ref[...], w2_ref[...],
                    preferred_element_type=jnp.float32)
        out_ref[...] = g * (1.0 / N) + b2_ref[...]


def _comb_call(acc2, hs, dinv_col, cvec_col, b1_row, W2, b2_row):
    nblk = NPAD // BLK
    return pl.pallas_call(
        _comb_body,
        grid=(nblk,),
        in_specs=[
            pl.BlockSpec((BLK, D), lambda i: (i, 0)),
            pl.BlockSpec((BLK, D), lambda i: (i + NPAD // BLK, 0)),
            pl.BlockSpec((BLK, D), lambda i: (i, 0)),
            pl.BlockSpec((BLK, 1), lambda i: (i, 0)),
            pl.BlockSpec((BLK, 1), lambda i: (i, 0)),
            pl.BlockSpec((1, D), lambda i: (0, 0)),
            pl.BlockSpec((D, D), lambda i: (0, 0)),
            pl.BlockSpec((1, D), lambda i: (0, 0)),
        ],
        out_specs=pl.BlockSpec((1, D), lambda i: (0, 0)),
        out_shape=jax.ShapeDtypeStruct((1, D), jnp.float32),
        scratch_shapes=[pltpu.VMEM((1, D), jnp.float32)],
    )(acc2, acc2, hs, dinv_col, cvec_col, b1_row, W2, b2_row)


def kernel(x, edge_index, W1, b1, W2, b2):
    deg_kernel, main_kernel = _build_sc_kernels()

    src = edge_index[0]
    dst = edge_index[1]
    pad_idx = (N + (jnp.arange(EXTRA, dtype=jnp.int32) % (NPAD - N))).astype(
        jnp.int32)
    src1d = jnp.concatenate([src, pad_idx])
    dst1d = jnp.concatenate([dst, pad_idx])
    x_pad = jnp.pad(x, ((0, NPAD - N), (0, 0)))

    ones_vec = jnp.ones((CLD,), jnp.float32)
    zrow = jnp.zeros((NPAD,), jnp.float32)
    zmat = jnp.zeros((NPAD, D), jnp.float32)

    h = _mm_call(x_pad, W1)
    deg2 = deg_kernel(dst1d, ones_vec, zrow)
    deg = deg2[:NPAD] + deg2[NPAD:] + 1.0
    dinv = lax.rsqrt(deg)

    hs = _scale_call(h, dinv.reshape(NPAD, 1))

    acc2, csum2 = main_kernel(src1d, dst1d, hs, dinv, zmat, zrow)

    csum = csum2[:NPAD] + csum2[NPAD:]
    cvec = dinv * (csum + dinv)
    cvec = jnp.where(jnp.arange(NPAD) < N, cvec, 0.0)

    return _comb_call(acc2, hs, dinv.reshape(NPAD, 1), cvec.reshape(NPAD, 1),
                      b1.reshape(1, D), W2, b2.reshape(1, D))


# speedup vs baseline: 48.9001x; 1.0101x over previous
import functools

import jax
import jax.numpy as jnp
from jax import lax
from jax.experimental import pallas as pl
from jax.experimental.pallas import tpu as pltpu
from jax.experimental.pallas import tpu_sc as plsc

N = 10000
E = 320000
D = 128

NC = 2
NS = 16
NW = NC * NS
NPAD = 10240
STRIPE = NPAD // NS
JROWS = 80
EROWS = NW * JROWS
E_PAD = EROWS * 128
EXTRA = E_PAD - E

EPT = JROWS * 128
CL = 160
CHT = EPT // CL
CLD = 512
NCHD = EPT // CLD

BLK = 640


@functools.lru_cache(maxsize=None)
def _build_sc_kernels():
    mesh = plsc.VectorSubcoreMesh(
        core_axis_name="c", subcore_axis_name="s",
        num_cores=NC, num_subcores=NS)

    @functools.partial(
        pl.kernel,
        out_type=jax.ShapeDtypeStruct((NC * NPAD,), jnp.float32),
        mesh=mesh,
        scratch_types=[
            pltpu.VMEM((EPT,), jnp.int32),
            pltpu.VMEM((CLD,), jnp.float32),
            pltpu.VMEM_SHARED((NPAD,), jnp.float32),
            pltpu.SemaphoreType.DMA,
        ],
    )
    def deg_kernel(dst_hbm, ones_hbm, zrow_hbm, out_hbm,
                   idx_v, ones_v, deg_sh, sem):
        c = lax.axis_index("c")
        s = lax.axis_index("s")
        w = c * NS + s
        r0 = s * STRIPE
        pltpu.sync_copy(ones_hbm, ones_v)
        pltpu.sync_copy(dst_hbm.at[pl.ds(w * EPT, EPT)], idx_v)
        pltpu.sync_copy(zrow_hbm.at[pl.ds(r0, STRIPE)],
                        deg_sh.at[pl.ds(r0, STRIPE)])
        plsc.subcore_barrier()

        def fire(k, carry):
            pltpu.async_copy(
                ones_v, deg_sh.at[idx_v.at[pl.ds(k * CLD, CLD)]], sem,
                add=True)
            return carry

        lax.fori_loop(0, NCHD, fire, 0)

        def drain(k, carry):
            pltpu.make_async_copy(
                ones_v, deg_sh.at[idx_v.at[pl.ds(k * CLD, CLD)]], sem).wait()
            return carry

        lax.fori_loop(0, NCHD, drain, 0)
        plsc.subcore_barrier()
        pltpu.sync_copy(deg_sh.at[pl.ds(r0, STRIPE)],
                        out_hbm.at[pl.ds(c * NPAD + r0, STRIPE)])

    @functools.partial(
        pl.kernel,
        out_type=(jax.ShapeDtypeStruct((NC * NPAD, D), jnp.float32),
                  jax.ShapeDtypeStruct((NC * NPAD,), jnp.float32)),
        mesh=mesh,
        scratch_types=[
            [pltpu.VMEM((CL,), jnp.int32) for _ in range(4)],
            [pltpu.VMEM((CL,), jnp.int32) for _ in range(4)],
            [pltpu.VMEM((CL, D), jnp.float32) for _ in range(2)],
            [pltpu.VMEM((CL,), jnp.float32) for _ in range(2)],
            pltpu.VMEM_SHARED((NPAD, D), jnp.float32),
            pltpu.VMEM_SHARED((NPAD,), jnp.float32),
            [pltpu.SemaphoreType.DMA for _ in range(4)],
            [pltpu.SemaphoreType.DMA for _ in range(2)],
            [pltpu.SemaphoreType.DMA for _ in range(2)],
            [pltpu.SemaphoreType.DMA for _ in range(2)],
            [pltpu.SemaphoreType.DMA for _ in range(2)],
        ],
    )
    def main_kernel(src_hbm, dst_hbm, hs_hbm, dinv_hbm, zmat_hbm, zrow_hbm,
                    acc_out, csum_out,
                    si, di, ubufs, dbufs, acc_sh, csum_sh,
                    isems, usems, dsems, asems, csems):
        c = lax.axis_index("c")
        s = lax.axis_index("s")
        w = c * NS + s
        r0 = s * STRIPE
        base = w * EPT

        def fire_idx(k, j):
            pltpu.async_copy(
                src_hbm.at[pl.ds(base + k * CL, CL)], si[j], isems[j])
            pltpu.async_copy(
                dst_hbm.at[pl.ds(base + k * CL, CL)], di[j], isems[j])

        def wait_idx(k, j):
            pltpu.make_async_copy(
                src_hbm.at[pl.ds(base + k * CL, CL)], si[j], isems[j]).wait()
            pltpu.make_async_copy(
                dst_hbm.at[pl.ds(base + k * CL, CL)], di[j], isems[j]).wait()

        def fire_gathers(j, b):
            pltpu.async_copy(hs_hbm.at[si[j]], ubufs[b], usems[b])
            pltpu.async_copy(dinv_hbm.at[di[j]], dbufs[b], dsems[b])

        def wait_gathers(j, b):
            pltpu.make_async_copy(hs_hbm.at[si[j]], ubufs[b], usems[b]).wait()
            pltpu.make_async_copy(
                dinv_hbm.at[di[j]], dbufs[b], dsems[b]).wait()

        def fire_scatters(j, b):
            pltpu.async_copy(ubufs[b], acc_sh.at[di[j]], asems[b], add=True)
            pltpu.async_copy(dbufs[b], csum_sh.at[si[j]], csems[b], add=True)

        def wait_scatters(j, b):
            pltpu.make_async_copy(ubufs[b], acc_sh.at[di[j]], asems[b]).wait()
            pltpu.make_async_copy(
                dbufs[b], csum_sh.at[si[j]], csems[b]).wait()

        for j in range(4):
            fire_idx(j, j)
        for b in range(2):
            wait_idx(b, b)
            fire_gathers(b, b)
        pltpu.sync_copy(zmat_hbm.at[pl.ds(r0, STRIPE)],
                        acc_sh.at[pl.ds(r0, STRIPE)])
        pltpu.sync_copy(zrow_hbm.at[pl.ds(r0, STRIPE)],
                        csum_sh.at[pl.ds(r0, STRIPE)])
        plsc.subcore_barrier()

        def quad(i, carry):
            for u in range(4):
                k = i * 4 + u
                j = u
                b = u % 2
                wait_gathers(j, b)
                fire_scatters(j, b)

                @pl.when(k + 2 < CHT)
                def _():
                    wait_scatters(j, b)
                    j2 = (u + 2) % 4
                    wait_idx(k + 2, j2)
                    fire_gathers(j2, b)

                    @pl.when(k + 4 < CHT)
                    def _():
                        fire_idx(k + 4, j)
            return carry

        lax.fori_loop(0, CHT // 4, quad, 0)
        wait_scatters(2, 0)
        wait_scatters(3, 1)
        plsc.subcore_barrier()
        pltpu.sync_copy(acc_sh.at[pl.ds(r0, STRIPE)],
                        acc_out.at[pl.ds(c * NPAD + r0, STRIPE)])
        pltpu.sync_copy(csum_sh.at[pl.ds(r0, STRIPE)],
                        csum_out.at[pl.ds(c * NPAD + r0, STRIPE)])

    return deg_kernel, main_kernel


def _mm_body(x_ref, w_ref, out_ref):
    out_ref[...] = jnp.dot(x_ref[...], w_ref[...],
                           preferred_element_type=jnp.float32)


def _mm_call(x_pad, W1):
    return pl.pallas_call(
        _mm_body,
        grid=(NPAD // BLK,),
        in_specs=[
            pl.BlockSpec((BLK, D), lambda i: (i, 0)),
            pl.BlockSpec((D, D), lambda i: (0, 0)),
        ],
        out_specs=pl.BlockSpec((BLK, D), lambda i: (i, 0)),
        out_shape=jax.ShapeDtypeStruct((NPAD, D), jnp.float32),
    )(x_pad, W1)


def _scale_body(h_ref, d_ref, out_ref):
    out_ref[...] = h_ref[...] * d_ref[...]


def _scale_call(h, dinv_col):
    return pl.pallas_call(
        _scale_body,
        grid=(NPAD // BLK,),
        in_specs=[
            pl.BlockSpec((BLK, D), lambda i: (i, 0)),
            pl.BlockSpec((BLK, 1), lambda i: (i, 0)),
        ],
        out_specs=pl.BlockSpec((BLK, D), lambda i: (i, 0)),
        out_shape=jax.ShapeDtypeStruct((NPAD, D), jnp.float32),
    )(h, dinv_col)


def _comb_body(a0_ref, a1_ref, hs_ref, d_ref, c_ref, b1_ref, w2_ref, b2_ref,
               out_ref, vacc_ref):
    i = pl.program_id(0)

    @pl.when(i == 0)
    def _():
        vacc_ref[...] = jnp.zeros((1, D), jnp.float32)

    a = a0_ref[...] + a1_ref[...] + hs_ref[...]
    r = jnp.maximum(d_ref[...] * a + b1_ref[...], 0.0)
    vacc_ref[...] += jnp.sum(c_ref[...] * r, axis=0, keepdims=True)

    @pl.when(i == pl.num_programs(0) - 1)
    def _():
        g = jnp.dot(vacc_ref[...], w2_ref[...],
                    preferred_element_type=jnp.float32)
        out_ref[...] = g * (1.0 / N) + b2_ref[...]


def _comb_call(acc2, hs, dinv_col, cvec_col, b1_row, W2, b2_row):
    nblk = NPAD // BLK
    return pl.pallas_call(
        _comb_body,
        grid=(nblk,),
        in_specs=[
            pl.BlockSpec((BLK, D), lambda i: (i, 0)),
            pl.BlockSpec((BLK, D), lambda i: (i + NPAD // BLK, 0)),
            pl.BlockSpec((BLK, D), lambda i: (i, 0)),
            pl.BlockSpec((BLK, 1), lambda i: (i, 0)),
            pl.BlockSpec((BLK, 1), lambda i: (i, 0)),
            pl.BlockSpec((1, D), lambda i: (0, 0)),
            pl.BlockSpec((D, D), lambda i: (0, 0)),
            pl.BlockSpec((1, D), lambda i: (0, 0)),
        ],
        out_specs=pl.BlockSpec((1, D), lambda i: (0, 0)),
        out_shape=jax.ShapeDtypeStruct((1, D), jnp.float32),
        scratch_shapes=[pltpu.VMEM((1, D), jnp.float32)],
    )(acc2, acc2, hs, dinv_col, cvec_col, b1_row, W2, b2_row)


def kernel(x, edge_index, W1, b1, W2, b2):
    deg_kernel, main_kernel = _build_sc_kernels()

    src = edge_index[0]
    dst = edge_index[1]
    pad_idx = (N + (jnp.arange(EXTRA, dtype=jnp.int32) % (NPAD - N))).astype(
        jnp.int32)
    src1d = jnp.concatenate([src, pad_idx])
    dst1d = jnp.concatenate([dst, pad_idx])
    x_pad = jnp.pad(x, ((0, NPAD - N), (0, 0)))

    ones_vec = jnp.ones((CLD,), jnp.float32)
    zrow = jnp.zeros((NPAD,), jnp.float32)
    zmat = jnp.zeros((NPAD, D), jnp.float32)

    h = _mm_call(x_pad, W1)
    deg2 = deg_kernel(dst1d, ones_vec, zrow)
    deg = deg2[:NPAD] + deg2[NPAD:] + 1.0
    dinv = lax.rsqrt(deg)

    hs = _scale_call(h, dinv.reshape(NPAD, 1))

    acc2, csum2 = main_kernel(src1d, dst1d, hs, dinv, zmat, zrow)

    csum = csum2[:NPAD] + csum2[NPAD:]
    cvec = dinv * (csum + dinv)
    cvec = jnp.where(jnp.arange(NPAD) < N, cvec, 0.0)

    return _comb_call(acc2, hs, dinv.reshape(NPAD, 1), cvec.reshape(NPAD, 1),
                      b1.reshape(1, D), W2, b2.reshape(1, D))
